# baseline (speedup 1.0000x reference)
import jax
import jax.numpy as jnp
from jax.experimental import pallas as pl
from jax.experimental.pallas import tpu as pltpu

_ROWS = 27278
_COLS = 1024
_BLOCK_ROWS = 512
_STEPS = (_ROWS + _BLOCK_ROWS - 1) // _BLOCK_ROWS


def _masked_mse_body(o_ref, t_ref, res_ref, acc_ref, cnt_ref):
    i = pl.program_id(0)

    @pl.when(i == 0)
    def _init():
        acc_ref[...] = jnp.zeros_like(acc_ref)
        cnt_ref[...] = jnp.zeros_like(cnt_ref)

    o = o_ref[...]
    t = t_ref[...]
    m = t != -1.0
    d = o - t

    @pl.when(i < _STEPS - 1)
    def _full():
        acc_ref[...] += jnp.where(m, d * d, 0.0)
        cnt_ref[...] += m.astype(jnp.float32)

    @pl.when(i == _STEPS - 1)
    def _tail():
        rows_left = _ROWS - (_STEPS - 1) * _BLOCK_ROWS
        valid = jax.lax.broadcasted_iota(
            jnp.int32, (_BLOCK_ROWS, _COLS), 0) < rows_left
        mv = jnp.logical_and(m, valid)
        acc_ref[...] += jnp.where(mv, d * d, 0.0)
        cnt_ref[...] += mv.astype(jnp.float32)
        res_ref[0, 0] = jnp.sum(acc_ref[...]) / jnp.sum(cnt_ref[...])


def kernel(output, target):
    spec = pl.BlockSpec((_BLOCK_ROWS, _COLS), lambda i: (i, 0))
    res = pl.pallas_call(
        _masked_mse_body,
        grid=(_STEPS,),
        in_specs=[spec, spec],
        out_specs=pl.BlockSpec(memory_space=pltpu.SMEM),
        out_shape=jax.ShapeDtypeStruct((1, 1), jnp.float32),
        scratch_shapes=[
            pltpu.VMEM((_BLOCK_ROWS, _COLS), jnp.float32),
            pltpu.VMEM((_BLOCK_ROWS, _COLS), jnp.float32),
        ],
    )(output.T, target.T)
    return res.reshape(())


# speedup vs baseline: 1.2115x; 1.2115x over previous
import jax
import jax.numpy as jnp
from jax.experimental import pallas as pl
from jax.experimental.pallas import tpu as pltpu

_ROWS = 27278
_COLS = 1024
_BLOCK_ROWS = 2048
_STEPS = (_ROWS + _BLOCK_ROWS - 1) // _BLOCK_ROWS


def _fold(x):
    return jnp.sum(x.reshape(_BLOCK_ROWS // 8, 8, _COLS), axis=0)


def _masked_mse_body(o_ref, t_ref, res_ref, acc_ref, cnt_ref):
    i = pl.program_id(0)

    @pl.when(i == 0)
    def _init():
        acc_ref[...] = jnp.zeros_like(acc_ref)
        cnt_ref[...] = jnp.zeros_like(cnt_ref)

    o = o_ref[...]
    t = t_ref[...]
    m = t != -1.0
    d = o - t

    @pl.when(i < _STEPS - 1)
    def _full():
        acc_ref[...] += _fold(jnp.where(m, d * d, 0.0))
        cnt_ref[...] += _fold(m.astype(jnp.float32))

    @pl.when(i == _STEPS - 1)
    def _tail():
        rows_left = _ROWS - (_STEPS - 1) * _BLOCK_ROWS
        valid = jax.lax.broadcasted_iota(
            jnp.int32, (_BLOCK_ROWS, _COLS), 0) < rows_left
        mv = jnp.logical_and(m, valid)
        acc_ref[...] += _fold(jnp.where(mv, d * d, 0.0))
        cnt_ref[...] += _fold(mv.astype(jnp.float32))
        res_ref[0, 0] = jnp.sum(acc_ref[...]) / jnp.sum(cnt_ref[...])


def kernel(output, target):
    spec = pl.BlockSpec((_BLOCK_ROWS, _COLS), lambda i: (i, 0))
    res = pl.pallas_call(
        _masked_mse_body,
        grid=(_STEPS,),
        in_specs=[spec, spec],
        out_specs=pl.BlockSpec(memory_space=pltpu.SMEM),
        out_shape=jax.ShapeDtypeStruct((1, 1), jnp.float32),
        scratch_shapes=[
            pltpu.VMEM((8, _COLS), jnp.float32),
            pltpu.VMEM((8, _COLS), jnp.float32),
        ],
    )(output.T, target.T)
    return res.reshape(())
